# baseline (speedup 1.0000x reference)
import functools

import jax
import jax.numpy as jnp
from jax import lax
from jax.experimental import pallas as pl
from jax.experimental.pallas import tpu as pltpu
from jax.experimental.pallas import tpu_sc as plsc

B, C, H, W = 2, 256, 320, 320
N = 512
HW = H * W
CHW = C * HW
NV = B * N
NWORK = 32
VPW = NV // NWORK
IDX_ROWS = VPW * C // 128
FIRE = 16


def _sc_gather(fm_flat, rows_arr, cols_arr):
    mesh = plsc.VectorSubcoreMesh(core_axis_name="c", subcore_axis_name="s")

    @functools.partial(
        pl.kernel,
        out_type=jax.ShapeDtypeStruct((NV * C // 128, 128), jnp.float32),
        mesh=mesh,
        scratch_types=[
            pltpu.VMEM((VPW,), jnp.int32),
            pltpu.VMEM((VPW,), jnp.int32),
            pltpu.VMEM((IDX_ROWS, 128), jnp.int32),
            pltpu.VMEM((IDX_ROWS, 128), jnp.float32),
            pltpu.SemaphoreType.DMA,
        ],
        compiler_params=pltpu.CompilerParams(needs_layout_passes=False),
    )
    def body(fm_hbm, rows_hbm, cols_hbm, out_hbm, rows_v, cols_v, idx_v,
             dat_v, sem):
        wid = lax.axis_index("s") * 2 + lax.axis_index("c")
        v0 = wid * VPW
        pltpu.sync_copy(rows_hbm.at[pl.ds(v0, VPW)], rows_v)
        pltpu.sync_copy(cols_hbm.at[pl.ds(v0, VPW)], cols_v)

        lane = jax.lax.iota(jnp.int32, 16)
        zeros = jnp.zeros((16,), jnp.int32)

        bases = []
        rowb = []
        for vc in range(VPW // 16):
            v_loc = vc * 16 + lane
            rows = rows_v[pl.ds(vc * 16, 16)]
            cols = cols_v[pl.ds(vc * 16, 16)]
            b = lax.shift_right_logical(v0 + v_loc, 9)
            bases.append(b * CHW + rows * W + cols)
            rowb.append(v_loc * (C // 128))

        def build(c, carry):
            coff = c * HW
            crow = lax.shift_right_logical(c, 7)
            ccol = (c & 127) + zeros
            for vc in range(VPW // 16):
                plsc.store_scatter(
                    idx_v, [rowb[vc] + crow, ccol], bases[vc] + coff
                )
            return carry

        lax.fori_loop(0, C, build, 0)

        def chunk(jb, carry):
            copies = []
            for i in range(FIRE):
                j = jb * FIRE + i
                copies.append(
                    pltpu.async_copy(fm_hbm.at[idx_v.at[j]], dat_v.at[j], sem)
                )
            for cp in copies:
                cp.wait()
            return carry

        lax.fori_loop(0, IDX_ROWS // FIRE, chunk, 0)

        pltpu.sync_copy(dat_v, out_hbm.at[pl.ds(wid * IDX_ROWS, IDX_ROWS)])

    return body(fm_flat, rows_arr, cols_arr)


def kernel(feature_map, vertices_positions):
    fm_flat = feature_map.reshape(-1)
    pos = vertices_positions.reshape(NV, 2).astype(jnp.int32)
    out = _sc_gather(fm_flat, pos[:, 0], pos[:, 1])
    return out.reshape(B, N, C)


# speedup vs baseline: 19.7986x; 19.7986x over previous
import functools

import jax
import jax.numpy as jnp
from jax import lax
from jax.experimental import pallas as pl
from jax.experimental.pallas import tpu as pltpu
from jax.experimental.pallas import tpu_sc as plsc

B, C, H, W = 2, 256, 320, 320
N = 512
NV = B * N
NWORK = 32
VPW = NV // NWORK
RPW = 2 * VPW
NROWS = NV * (C // 128)


def _sc_gather(fm_rows, rows_arr, cols_arr):
    mesh = plsc.VectorSubcoreMesh(core_axis_name="c", subcore_axis_name="s")

    @functools.partial(
        pl.kernel,
        out_type=jax.ShapeDtypeStruct((NROWS, 128), jnp.float32),
        mesh=mesh,
        scratch_types=[
            pltpu.VMEM((VPW,), jnp.int32),
            pltpu.VMEM((VPW,), jnp.int32),
            pltpu.VMEM((RPW,), jnp.int32),
            pltpu.VMEM((RPW, 128), jnp.float32),
            pltpu.SemaphoreType.DMA,
        ],
        compiler_params=pltpu.CompilerParams(needs_layout_passes=False),
    )
    def body(fm_hbm, rows_hbm, cols_hbm, out_hbm, rows_v, cols_v, idx_v,
             dat_v, sem):
        wid = lax.axis_index("s") * 2 + lax.axis_index("c")
        v0 = wid * VPW
        pltpu.sync_copy(rows_hbm.at[pl.ds(v0, VPW)], rows_v)
        pltpu.sync_copy(cols_hbm.at[pl.ds(v0, VPW)], cols_v)

        lane = jax.lax.iota(jnp.int32, 16)
        for vc in range(VPW // 16):
            r = rows_v[pl.ds(vc * 16, 16)]
            c = cols_v[pl.ds(vc * 16, 16)]
            b = lax.shift_right_logical(v0 + vc * 16 + lane, 9)
            rho0 = ((b * H + r) * (W // 8) + lax.shift_right_logical(c, 3)) \
                * 16 + (c & 7)
            pos = (vc * 16 + lane) * 2
            plsc.store_scatter(idx_v, [pos], rho0)
            plsc.store_scatter(idx_v, [pos + 1], rho0 + 8)

        pltpu.async_copy(fm_hbm.at[idx_v], dat_v, sem).wait()
        pltpu.sync_copy(dat_v, out_hbm.at[pl.ds(wid * RPW, RPW)])

    return body(fm_rows, rows_arr, cols_arr)


def kernel(feature_map, vertices_positions):
    fm_rows = (
        feature_map.transpose(0, 2, 3, 1)
        .reshape(B, H, W // 8, 8, C // 128, 128)
        .transpose(0, 1, 2, 4, 3, 5)
        .reshape(B * H * (W // 8) * (C // 128) * 8, 128)
    )
    pos = vertices_positions.reshape(NV, 2).astype(jnp.int32)
    out = _sc_gather(fm_rows, pos[:, 0], pos[:, 1])
    return out.reshape(B, N, C)
